# baseline (speedup 1.0000x reference)
import functools

import jax
import jax.numpy as jnp
from jax import lax
from jax.experimental import pallas as pl
from jax.experimental.pallas import tpu as pltpu
from jax.experimental.pallas import tpu_sc as plsc

_B, _S, _H, _D = 1024, 200, 8, 32
_NPH = 100000
_NB = 32
_NBC = _B // _NB
_IPW = _S // 2
_NSL = _NB * _H
_NBUF = 4

_mesh = plsc.VectorSubcoreMesh(core_axis_name="c", subcore_axis_name="s")


@functools.partial(
    pl.kernel,
    out_type=jax.ShapeDtypeStruct((_S, _B * _H, _D), jnp.float32),
    mesh=_mesh,
    compiler_params=pltpu.CompilerParams(
        use_tc_tiling_on_sc=False, needs_layout_passes=False),
    scratch_types=[
        pltpu.VMEM((_NBUF, _NB, 16), jnp.int32),
        pltpu.VMEM((_NBUF, 2, 2, 128), jnp.int32),
        pltpu.VMEM((_NBUF, 2, _NSL, _D), jnp.float32),
        pltpu.SemaphoreType.DMA,
        pltpu.SemaphoreType.DMA,
        pltpu.SemaphoreType.DMA,
        pltpu.SemaphoreType.DMA,
        pltpu.SemaphoreType.DMA,
        pltpu.SemaphoreType.DMA,
        pltpu.SemaphoreType.DMA,
        pltpu.SemaphoreType.DMA,
        pltpu.SemaphoreType.DMA,
        pltpu.SemaphoreType.DMA,
        pltpu.SemaphoreType.DMA,
        pltpu.SemaphoreType.DMA,
    ],
)
def _mhe_kernel(ids_hbm, table_hbm, out_hbm, raw_v, gidx_v, rows_v,
                sem_i0, sem_i1, sem_i2, sem_i3,
                sem_g0, sem_g1, sem_g2, sem_g3,
                sem_w0, sem_w1, sem_w2, sem_w3):
    wid = lax.axis_index("s") * 2 + lax.axis_index("c")
    b0 = wid * _NB
    o0 = wid * _NSL

    sem_i = (sem_i0, sem_i1, sem_i2, sem_i3)
    sem_g = (sem_g0, sem_g1, sem_g2, sem_g3)
    sem_w = (sem_w0, sem_w1, sem_w2, sem_w3)

    iota = lax.iota(jnp.int32, 16)
    rv = iota >> 3
    cv = iota & 7
    cv8 = cv + 8
    offv = cv * _NPH

    def idx_copy(item, slot):
        return pltpu.make_async_copy(
            ids_hbm.at[pl.ds(b0, _NB), pl.ds(item * 16, 16)],
            raw_v.at[slot], sem_i[slot])

    def write_copy(item, slot, sl):
        return pltpu.make_async_copy(
            rows_v.at[slot, sl],
            out_hbm.at[item * 2 + sl, pl.ds(o0, _NSL)], sem_w[slot])

    def gather_drain(item, slot, sl):
        return pltpu.make_async_copy(
            out_hbm.at[item * 2 + sl, pl.ds(o0, _NSL)],
            rows_v.at[slot, sl], sem_g[slot])

    def stage(slot):
        rowv = rv
        for g in range(2):
            for k in range(8):
                v0 = plsc.load_gather(raw_v.at[slot], [rowv, cv]) + offv
                gidx_v[slot, 0, g, pl.ds(16 * k, 16)] = v0
                v1 = plsc.load_gather(raw_v.at[slot], [rowv, cv8]) + offv
                gidx_v[slot, 1, g, pl.ds(16 * k, 16)] = v1
                rowv = rowv + 2

    for slot in range(_NBUF):
        idx_copy(slot, slot).start()

    @pl.loop(0, _IPW, step=_NBUF)
    def _item_quad(i0):
        for slot in range(_NBUF):
            it = i0 + slot
            sp = (slot - 1) % _NBUF
            idx_copy(it, slot).wait()
            stage(slot)

            @pl.when(it + _NBUF < _IPW)
            def _():
                idx_copy(it + _NBUF, slot).start()

            @pl.when(it >= _NBUF)
            def _():
                write_copy(it - _NBUF, slot, 0).wait()
                write_copy(it - _NBUF, slot, 1).wait()

            for sl in range(2):
                for g in range(2):
                    pltpu.make_async_copy(
                        table_hbm.at[gidx_v.at[slot, sl, g]],
                        rows_v.at[slot, sl, pl.ds(g * 128, 128)],
                        sem_g[slot]).start()

            @pl.when(it >= 1)
            def _():
                gather_drain(it - 1, sp, 0).wait()
                gather_drain(it - 1, sp, 1).wait()
                write_copy(it - 1, sp, 0).start()
                write_copy(it - 1, sp, 1).start()

    last = _IPW - 1
    lslot = last % _NBUF
    gather_drain(last, lslot, 0).wait()
    gather_drain(last, lslot, 1).wait()
    write_copy(last, lslot, 0).start()
    write_copy(last, lslot, 1).start()
    for k in range(_NBUF):
        it = _IPW - _NBUF + k
        write_copy(it, it % _NBUF, 0).wait()
        write_copy(it, it % _NBUF, 1).wait()


_V = _H * _NPH
_TC = 896
_NFULL = _V // _TC
_TREM = _V - _NFULL * _TC
_TIPW = 28


@functools.partial(
    pl.kernel,
    out_type=jax.ShapeDtypeStruct((_V * _D,), jnp.float32),
    mesh=_mesh,
    compiler_params=pltpu.CompilerParams(
        use_tc_tiling_on_sc=True, needs_layout_passes=False),
    scratch_types=[
        pltpu.VMEM((2, _D, _TC), jnp.float32),
        pltpu.VMEM((2, _TC * _D), jnp.float32),
        pltpu.SemaphoreType.DMA,
        pltpu.SemaphoreType.DMA,
        pltpu.SemaphoreType.DMA,
    ],
)
def _tr_kernel(tblT_hbm, out_hbm, inb_v, outb_v, sem_i, sem_o0, sem_o1):
    wid = lax.axis_index("s") * 2 + lax.axis_index("c")
    sem_o = (sem_o0, sem_o1)
    iota = lax.iota(jnp.int32, 16)
    iota16 = iota + 16

    def in_copy(c, slot):
        col0 = pl.multiple_of(c * _TC, 128)
        return pltpu.make_async_copy(
            tblT_hbm.at[:, pl.ds(col0, _TC)], inb_v.at[slot], sem_i)

    def out_copy(c, slot):
        o0 = pl.multiple_of(c * (_TC * _D), 1024)
        return pltpu.make_async_copy(
            outb_v.at[slot], out_hbm.at[pl.ds(o0, _TC * _D)], sem_o[slot])

    def transpose_chunk(slot, ncols):
        @pl.loop(0, ncols, unroll=8)
        def _col(cc):
            csp = jnp.full((16,), cc, jnp.int32)
            lo = plsc.load_gather(inb_v.at[slot], [iota, csp])
            hi = plsc.load_gather(inb_v.at[slot], [iota16, csp])
            outb_v[slot, pl.ds(cc * _D, 16)] = lo
            outb_v[slot, pl.ds(cc * _D + 16, 16)] = hi

    @pl.loop(0, _TIPW, step=2)
    def _chunk(i0):
        for slot in range(2):
            i = i0 + slot
            c = wid + 32 * i

            @pl.when(c < _NFULL)
            def _():
                @pl.when(i >= 2)
                def _():
                    out_copy(c - 64, slot).wait()

                in_copy(c, slot).start()
                in_copy(c, slot).wait()
                transpose_chunk(slot, _TC)
                out_copy(c, slot).start()

    out_copy(0, 0).wait()
    out_copy(0, 1).wait()

    @pl.when(wid == 31)
    def _():
        col0 = _NFULL * _TC
        cp_in = pltpu.make_async_copy(
            tblT_hbm.at[:, pl.ds(col0, _TREM)],
            inb_v.at[0, :, pl.ds(0, _TREM)], sem_i)
        cp_in.start()
        cp_in.wait()
        transpose_chunk(0, _TREM)
        cp_out = pltpu.make_async_copy(
            outb_v.at[0, pl.ds(0, _TREM * _D)],
            out_hbm.at[pl.ds(col0 * _D, _TREM * _D)], sem_o0)
        cp_out.start()
        cp_out.wait()


def kernel(input_ids, table):
    ids2 = input_ids.reshape(_B, _S * _H)
    tbl_lin = _tr_kernel(jnp.transpose(table, (1, 0)))
    tbl = tbl_lin.reshape(_V, _D)
    out = _mhe_kernel(ids2, tbl)
    return out.reshape(_S, _B, _H, _D)


# speedup vs baseline: 1.0339x; 1.0339x over previous
import functools

import jax
import jax.numpy as jnp
from jax import lax
from jax.experimental import pallas as pl
from jax.experimental.pallas import tpu as pltpu
from jax.experimental.pallas import tpu_sc as plsc

_B, _S, _H, _D = 1024, 200, 8, 32
_NPH = 100000
_NB = 32
_NBC = _B // _NB
_IPW = _S // 2
_NSL = _NB * _H
_NBUF = 4

_mesh = plsc.VectorSubcoreMesh(core_axis_name="c", subcore_axis_name="s")


@functools.partial(
    pl.kernel,
    out_type=jax.ShapeDtypeStruct((_S, _B * _H, _D), jnp.float32),
    mesh=_mesh,
    compiler_params=pltpu.CompilerParams(
        use_tc_tiling_on_sc=False, needs_layout_passes=False),
    scratch_types=[
        pltpu.VMEM((_NBUF, _NB, 16), jnp.int32),
        pltpu.VMEM((_NBUF, 2, 2, 128), jnp.int32),
        pltpu.VMEM((_NBUF, 2, _NSL, _D), jnp.float32),
        pltpu.SemaphoreType.DMA,
        pltpu.SemaphoreType.DMA,
        pltpu.SemaphoreType.DMA,
        pltpu.SemaphoreType.DMA,
        pltpu.SemaphoreType.DMA,
        pltpu.SemaphoreType.DMA,
        pltpu.SemaphoreType.DMA,
        pltpu.SemaphoreType.DMA,
        pltpu.SemaphoreType.DMA,
        pltpu.SemaphoreType.DMA,
        pltpu.SemaphoreType.DMA,
        pltpu.SemaphoreType.DMA,
    ],
)
def _mhe_kernel(ids_hbm, table_hbm, out_hbm, raw_v, gidx_v, rows_v,
                sem_i0, sem_i1, sem_i2, sem_i3,
                sem_g0, sem_g1, sem_g2, sem_g3,
                sem_w0, sem_w1, sem_w2, sem_w3):
    wid = lax.axis_index("s") * 2 + lax.axis_index("c")
    b0 = wid * _NB
    o0 = wid * _NSL

    sem_i = (sem_i0, sem_i1, sem_i2, sem_i3)
    sem_g = (sem_g0, sem_g1, sem_g2, sem_g3)
    sem_w = (sem_w0, sem_w1, sem_w2, sem_w3)

    iota = lax.iota(jnp.int32, 16)
    rv = iota >> 3
    cv = iota & 7
    cv8 = cv + 8
    offv = cv * _NPH

    def idx_copy(item, slot):
        return pltpu.make_async_copy(
            ids_hbm.at[pl.ds(b0, _NB), pl.ds(item * 16, 16)],
            raw_v.at[slot], sem_i[slot])

    def write_copy(item, slot, sl):
        return pltpu.make_async_copy(
            rows_v.at[slot, sl],
            out_hbm.at[item * 2 + sl, pl.ds(o0, _NSL)], sem_w[slot])

    def gather_drain(item, slot, sl):
        return pltpu.make_async_copy(
            out_hbm.at[item * 2 + sl, pl.ds(o0, _NSL)],
            rows_v.at[slot, sl], sem_g[slot])

    def stage(slot):
        rowv = rv
        for g in range(2):
            for k in range(8):
                v0 = plsc.load_gather(raw_v.at[slot], [rowv, cv]) + offv
                gidx_v[slot, 0, g, pl.ds(16 * k, 16)] = v0
                v1 = plsc.load_gather(raw_v.at[slot], [rowv, cv8]) + offv
                gidx_v[slot, 1, g, pl.ds(16 * k, 16)] = v1
                rowv = rowv + 2

    for slot in range(_NBUF):
        idx_copy(slot, slot).start()

    @pl.loop(0, _IPW, step=_NBUF)
    def _item_quad(i0):
        for slot in range(_NBUF):
            it = i0 + slot
            sp = (slot - 1) % _NBUF
            idx_copy(it, slot).wait()
            stage(slot)

            @pl.when(it + _NBUF < _IPW)
            def _():
                idx_copy(it + _NBUF, slot).start()

            @pl.when(it >= _NBUF)
            def _():
                write_copy(it - _NBUF, slot, 0).wait()
                write_copy(it - _NBUF, slot, 1).wait()

            for sl in range(2):
                for g in range(2):
                    pltpu.make_async_copy(
                        table_hbm.at[gidx_v.at[slot, sl, g]],
                        rows_v.at[slot, sl, pl.ds(g * 128, 128)],
                        sem_g[slot]).start()

            @pl.when(it >= 1)
            def _():
                gather_drain(it - 1, sp, 0).wait()
                gather_drain(it - 1, sp, 1).wait()
                write_copy(it - 1, sp, 0).start()
                write_copy(it - 1, sp, 1).start()

    last = _IPW - 1
    lslot = last % _NBUF
    gather_drain(last, lslot, 0).wait()
    gather_drain(last, lslot, 1).wait()
    write_copy(last, lslot, 0).start()
    write_copy(last, lslot, 1).start()
    for k in range(_NBUF):
        it = _IPW - _NBUF + k
        write_copy(it, it % _NBUF, 0).wait()
        write_copy(it, it % _NBUF, 1).wait()


_V = _H * _NPH
_TC = 896
_NFULL = _V // _TC
_TREM = _V - _NFULL * _TC
_TIPW = 28


@functools.partial(
    pl.kernel,
    out_type=jax.ShapeDtypeStruct((_V * _D,), jnp.float32),
    mesh=_mesh,
    compiler_params=pltpu.CompilerParams(
        use_tc_tiling_on_sc=True, needs_layout_passes=False),
    scratch_types=[
        pltpu.VMEM((2, _D, _TC + 1), jnp.float32),
        pltpu.VMEM((2, _TC * _D), jnp.float32),
        pltpu.SemaphoreType.DMA,
        pltpu.SemaphoreType.DMA,
        pltpu.SemaphoreType.DMA,
        pltpu.SemaphoreType.DMA,
    ],
)
def _tr_kernel(tblT_hbm, out_hbm, inb_v, outb_v, sem_i0, sem_i1,
               sem_o0, sem_o1):
    wid = lax.axis_index("s") * 2 + lax.axis_index("c")
    sem_i = (sem_i0, sem_i1)
    sem_o = (sem_o0, sem_o1)
    iota = lax.iota(jnp.int32, 16)
    iota16 = iota + 16

    def in_copy(c, slot):
        col0 = pl.multiple_of(c * _TC, 128)
        return pltpu.make_async_copy(
            tblT_hbm.at[:, pl.ds(col0, _TC)],
            inb_v.at[slot, :, pl.ds(0, _TC)], sem_i[slot])

    def out_copy(c, slot):
        o0 = pl.multiple_of(c * (_TC * _D), 1024)
        return pltpu.make_async_copy(
            outb_v.at[slot], out_hbm.at[pl.ds(o0, _TC * _D)], sem_o[slot])

    def transpose_chunk(slot, ncols):
        @pl.loop(0, ncols, unroll=8)
        def _col(cc):
            csp = jnp.full((16,), cc, jnp.int32)
            lo = plsc.load_gather(inb_v.at[slot], [iota, csp])
            hi = plsc.load_gather(inb_v.at[slot], [iota16, csp])
            outb_v[slot, pl.ds(cc * _D, 16)] = lo
            outb_v[slot, pl.ds(cc * _D + 16, 16)] = hi

    in_copy(wid, 0).start()
    in_copy(wid + 32, 1).start()

    @pl.loop(0, _TIPW, step=2)
    def _chunk(i0):
        for slot in range(2):
            i = i0 + slot
            c = wid + 32 * i

            @pl.when(c < _NFULL)
            def _():
                in_copy(c, slot).wait()

                @pl.when(i >= 2)
                def _():
                    out_copy(c - 64, slot).wait()

                transpose_chunk(slot, _TC)

                @pl.when(c + 64 < _NFULL)
                def _():
                    in_copy(c + 64, slot).start()

                out_copy(c, slot).start()

    out_copy(0, 0).wait()
    out_copy(0, 1).wait()

    @pl.when(wid == 31)
    def _():
        col0 = _NFULL * _TC
        cp_in = pltpu.make_async_copy(
            tblT_hbm.at[:, pl.ds(col0, _TREM)],
            inb_v.at[0, :, pl.ds(0, _TREM)], sem_i0)
        cp_in.start()
        cp_in.wait()
        transpose_chunk(0, _TREM)
        cp_out = pltpu.make_async_copy(
            outb_v.at[0, pl.ds(0, _TREM * _D)],
            out_hbm.at[pl.ds(col0 * _D, _TREM * _D)], sem_o0)
        cp_out.start()
        cp_out.wait()


def kernel(input_ids, table):
    ids2 = input_ids.reshape(_B, _S * _H)
    tbl_lin = _tr_kernel(jnp.transpose(table, (1, 0)))
    tbl = tbl_lin.reshape(_V, _D)
    out = _mhe_kernel(ids2, tbl)
    return out.reshape(_S, _B, _H, _D)
